# baseline (speedup 1.0000x reference)
import functools

import jax
import jax.numpy as jnp
from jax import lax
from jax.experimental import pallas as pl
from jax.experimental.pallas import tpu as pltpu
from jax.experimental.pallas import tpu_sc as plsc

NC = 2
NS = 16
NW = NC * NS
SR = 8


def _sc_body(B, N, C, K, dist_hbm, idxk_hbm, r_hbm, x_hbm,
             med_hbm, xr_hbm,
             idxk_v, r_v, xs_v, med_v, xr_v,
             slab0, slab1, sem_a, sem_b):
    BN = B * N
    PW = BN // NW
    wid = lax.axis_index("c") * NS + lax.axis_index("s")
    base = wid * PW
    b = base // N
    n0 = base - b * N

    pltpu.sync_copy(idxk_hbm.at[pl.ds(base * K, PW * K)], idxk_v)
    pltpu.sync_copy(r_hbm.at[pl.ds(base, PW)], r_v)
    pltpu.sync_copy(x_hbm.at[pl.ds(b * N * C, N * C)], xs_v)

    lane = lax.iota(jnp.int32, 16)

    def mk_xr(t, carry):
        pvec = t * 16 + lane
        rvec = r_v[pl.ds(t * 16, 16)]
        g = plsc.load_gather(idxk_v, [pvec * K + rvec])
        for c in range(C):
            xr_v[pl.ds(c * PW + t * 16, 16)] = plsc.load_gather(
                xs_v, [g * C + c])
        return carry
    lax.fori_loop(0, PW // 16, mk_xr, 0, unroll=False)

    NSLAB = PW // SR
    slabs = (slab0, slab1)
    sems = (sem_a, sem_b)

    def slab_copy(g, slot):
        return pltpu.make_async_copy(
            dist_hbm.at[b, pl.ds(n0 + g * SR, SR)], slabs[slot], sems[slot])

    def process(g, slot, lane0, mv):
        sl = slabs[slot]
        for u in range(SR):
            p = g * SR + u
            iv0 = idxk_v[pl.ds(p * K, 16)]
            iv1 = idxk_v[pl.ds(p * K + 16, 16)]
            urow = jnp.full((16,), u, jnp.int32)
            a = plsc.load_gather(sl, [urow, iv0])
            bb = plsc.load_gather(sl, [urow, iv1])
            lo = jnp.minimum(jnp.sort(a), jnp.flip(jnp.sort(bb)))
            mv = jnp.where(lane == lane0 + u, jnp.max(lo), mv)
        return mv

    slab_copy(0, 0).start()

    def pair(i, carry):
        g0 = 2 * i
        g1 = 2 * i + 1
        slab_copy(g1, 1).start()
        slab_copy(g0, 0).wait()
        mv = process(g0, 0, 0, jnp.zeros((16,), jnp.float32))

        @pl.when(g0 + 2 < NSLAB)
        def _():
            slab_copy(g0 + 2, 0).start()

        slab_copy(g1, 1).wait()
        mv = process(g1, 1, 8, mv)
        med_v[pl.ds(i * 16, 16)] = mv
        return carry
    lax.fori_loop(0, NSLAB // 2, pair, 0, unroll=False)

    pltpu.sync_copy(med_v, med_hbm.at[pl.ds(base, PW)])
    for c in range(C):
        pltpu.sync_copy(xr_v.at[pl.ds(c * PW, PW)],
                        xr_hbm.at[pl.ds(c * BN + base, PW)])


def _sc_call(B, N, C, K, distance, idx_k, r_flat, x):
    BN = B * N
    PW = BN // NW
    mesh = plsc.VectorSubcoreMesh(core_axis_name="c", subcore_axis_name="s")
    kern = pl.kernel(
        functools.partial(_sc_body, B, N, C, K),
        out_type=(
            jax.ShapeDtypeStruct((BN,), jnp.float32),
            jax.ShapeDtypeStruct((C * BN,), jnp.float32),
        ),
        mesh=mesh,
        compiler_params=pltpu.CompilerParams(needs_layout_passes=False),
        scratch_types=[
            pltpu.VMEM((PW * K,), jnp.int32),
            pltpu.VMEM((PW,), jnp.int32),
            pltpu.VMEM((N * C,), jnp.float32),
            pltpu.VMEM((PW,), jnp.float32),
            pltpu.VMEM((C * PW,), jnp.float32),
            pltpu.VMEM((SR, N), jnp.float32),
            pltpu.VMEM((SR, N), jnp.float32),
            pltpu.SemaphoreType.DMA,
            pltpu.SemaphoreType.DMA,
        ],
    )
    return kern(distance, idx_k, r_flat, x)


def _tc_body(x2_ref, nt2_ref, xr_ref, med_ref, out_ref):
    eye = jnp.eye(3, dtype=jnp.float32)
    cdims = (((1,), (1,)), ((), ()))
    xt = lax.dot_general(eye, x2_ref[...], cdims,
                         preferred_element_type=jnp.float32)
    nt = lax.dot_general(eye, nt2_ref[...], cdims,
                         preferred_element_type=jnp.float32)
    xv = xr_ref[...] - xt
    dot = jnp.sum(xv * nt, axis=0, keepdims=True)
    xp = xv - dot * nt
    n2 = jnp.sum(xp * xp, axis=0, keepdims=True)
    norms = jnp.maximum(jnp.sqrt(n2), 1e-6)
    half = norms * 0.5
    mk = jnp.sqrt(med_ref[...])
    clamped = jnp.where(half > mk, mk, half)
    outp = xt + xp * (clamped / norms)
    out_ref[...] = lax.dot_general(outp, eye, (((0,), (0,)), ((), ())),
                                   preferred_element_type=jnp.float32)


def _tc_call(x2, nt2, xr, med):
    BN, C = x2.shape
    return pl.pallas_call(
        _tc_body,
        out_shape=jax.ShapeDtypeStruct((BN, C), jnp.float32),
    )(x2, nt2, xr, med)


def kernel(x, distance, idx_k, normals):
    B, N, C = x.shape
    K = idx_k.shape[-1]
    BN = B * N
    r = jax.random.randint(jax.random.key(42), (B, N, 1), 0, K,
                           dtype=jnp.int32)
    med, xr = _sc_call(B, N, C, K, distance, idx_k.reshape(BN * K),
                       r.reshape(BN), x.reshape(BN * C))
    out2 = _tc_call(x.reshape(BN, C), normals.reshape(BN, C),
                    xr.reshape(C, BN), med.reshape(1, BN))
    return out2.reshape(B, N, C)


# speedup vs baseline: 1.3581x; 1.3581x over previous
import functools

import jax
import jax.numpy as jnp
from jax import lax
from jax.experimental import pallas as pl
from jax.experimental.pallas import tpu as pltpu
from jax.experimental.pallas import tpu_sc as plsc

NC = 2
NS = 16
NW = NC * NS


def _sc_body(B, N, C, K, dist_hbm, idxk_hbm, r_hbm, xt_hbm,
             med_hbm, xr_hbm,
             idxk_v, gb0, gb1, gb2, gb3, xs_v, r_v, med_v, xr_v,
             sem_a, sem_b, sem_c, sem_d):
    BN = B * N
    PW = BN // NW
    wid = lax.axis_index("c") * NS + lax.axis_index("s")
    base = wid * PW
    b = base // N
    n0 = base - b * N

    pltpu.sync_copy(idxk_hbm.at[pl.ds(base * K, PW * K)], idxk_v)
    pltpu.sync_copy(r_hbm.at[pl.ds(base, PW)], r_v)
    for c in range(C):
        pltpu.sync_copy(xt_hbm.at[pl.ds(c * BN + b * N, N)],
                        xs_v.at[pl.ds(c * N, N)])

    lane = lax.iota(jnp.int32, 16)

    def mk_xr(t, carry):
        pvec = t * 16 + lane
        rvec = r_v[pl.ds(t * 16, 16)]
        g = plsc.load_gather(idxk_v, [pvec * K + rvec])
        for c in range(C):
            xr_v[pl.ds(c * PW + t * 16, 16)] = plsc.load_gather(
                xs_v, [g + c * N])
        return carry
    lax.fori_loop(0, PW // 16, mk_xr, 0, unroll=False)

    SR = 8
    NSLAB = PW // SR
    NBUF = 4
    slabs = (gb0, gb1, gb2, gb3)
    sems = (sem_a, sem_b, sem_c, sem_d)

    def slab_copy(g, slot):
        return pltpu.make_async_copy(
            dist_hbm.at[b, pl.ds(n0 + g * SR, SR)], slabs[slot], sems[slot])

    def process(g, slot, lane0, mv):
        sl = slabs[slot]
        for u in range(SR):
            p = g * SR + u
            iv0 = idxk_v[pl.ds(p * K, 16)]
            iv1 = idxk_v[pl.ds(p * K + 16, 16)]
            urow = jnp.full((16,), u, jnp.int32)
            a = plsc.load_gather(sl, [urow, iv0])
            bb = plsc.load_gather(sl, [urow, iv1])
            lo = jnp.minimum(jnp.sort(a), jnp.flip(jnp.sort(bb)))
            mv = jnp.where(lane == lane0 + u, jnp.max(lo), mv)
        return mv

    for s in range(NBUF):
        slab_copy(s, s).start()

    def quad(i, carry):
        mv = jnp.zeros((16,), jnp.float32)
        for q in range(NBUF):
            g = i * NBUF + q
            slab_copy(g, q).wait()
            mv = process(g, q, 8 * (q % 2), mv)

            @pl.when(g + NBUF < NSLAB)
            def _():
                slab_copy(g + NBUF, q).start()

            if q % 2 == 1:
                med_v[pl.ds((i * 2 + q // 2) * 16, 16)] = mv
                mv = jnp.zeros((16,), jnp.float32)
        return carry
    lax.fori_loop(0, NSLAB // NBUF, quad, 0, unroll=False)

    pltpu.sync_copy(med_v, med_hbm.at[pl.ds(base, PW)])
    for c in range(C):
        pltpu.sync_copy(xr_v.at[pl.ds(c * PW, PW)],
                        xr_hbm.at[pl.ds(c * BN + base, PW)])


def _sc_call(B, N, C, K, distance, idxk_flat, r_flat, xt_flat):
    BN = B * N
    PW = BN // NW
    mesh = plsc.VectorSubcoreMesh(core_axis_name="c", subcore_axis_name="s")
    kern = pl.kernel(
        functools.partial(_sc_body, B, N, C, K),
        out_type=(
            jax.ShapeDtypeStruct((BN,), jnp.float32),
            jax.ShapeDtypeStruct((C * BN,), jnp.float32),
        ),
        mesh=mesh,
        compiler_params=pltpu.CompilerParams(needs_layout_passes=False),
        scratch_types=[
            pltpu.VMEM((PW * K,), jnp.int32),
            pltpu.VMEM((8, N), jnp.float32),
            pltpu.VMEM((8, N), jnp.float32),
            pltpu.VMEM((8, N), jnp.float32),
            pltpu.VMEM((8, N), jnp.float32),
            pltpu.VMEM((C * N,), jnp.float32),
            pltpu.VMEM((PW,), jnp.int32),
            pltpu.VMEM((PW,), jnp.float32),
            pltpu.VMEM((C * PW,), jnp.float32),
            pltpu.SemaphoreType.DMA,
            pltpu.SemaphoreType.DMA,
            pltpu.SemaphoreType.DMA,
            pltpu.SemaphoreType.DMA,
        ],
    )
    return kern(distance, idxk_flat, r_flat, xt_flat)


def _tc_body(xt_ref, nt_ref, xr_ref, med_ref, out_ref):
    xt = xt_ref[...]
    nt = nt_ref[...]
    xv = xr_ref[...] - xt
    dot = jnp.sum(xv * nt, axis=0, keepdims=True)
    xp = xv - dot * nt
    n2 = jnp.sum(xp * xp, axis=0, keepdims=True)
    norms = jnp.maximum(jnp.sqrt(n2), 1e-6)
    half = norms * 0.5
    mk = jnp.sqrt(med_ref[...])
    clamped = jnp.where(half > mk, mk, half)
    out_ref[...] = xt + xp * (clamped / norms)


def _tc_call(xt, nt, xr, med):
    C, BN = xt.shape
    return pl.pallas_call(
        _tc_body,
        out_shape=jax.ShapeDtypeStruct((C, BN), jnp.float32),
    )(xt, nt, xr, med)


def kernel(x, distance, idx_k, normals):
    B, N, C = x.shape
    K = idx_k.shape[-1]
    BN = B * N
    r = jax.random.randint(jax.random.key(42), (B, N, 1), 0, K,
                           dtype=jnp.int32)
    xt = x.transpose(2, 0, 1).reshape(C, BN)
    nt = normals.transpose(2, 0, 1).reshape(C, BN)
    med, xr = _sc_call(B, N, C, K, distance, idx_k.reshape(BN * K),
                       r.reshape(BN), xt.reshape(C * BN))
    out_t = _tc_call(xt, nt, xr.reshape(C, BN), med.reshape(1, BN))
    return out_t.reshape(C, B, N).transpose(1, 2, 0)


# speedup vs baseline: 1.3718x; 1.0101x over previous
import functools

import jax
import jax.numpy as jnp
from jax import lax
from jax.experimental import pallas as pl
from jax.experimental.pallas import tpu as pltpu
from jax.experimental.pallas import tpu_sc as plsc

NC = 2
NS = 16
NW = NC * NS


def _sc_body(B, N, C, K, dist_hbm, idxk_hbm, r_hbm, xt_hbm,
             med_hbm, xr_hbm,
             idxk_v, gb0, gb1, gb2, gb3, xs_v, r_v, med_v, xr_v,
             sem_a, sem_b, sem_c, sem_d):
    BN = B * N
    PW = BN // NW
    wid = lax.axis_index("c") * NS + lax.axis_index("s")
    base = wid * PW
    b = base // N
    n0 = base - b * N

    lane = lax.iota(jnp.int32, 16)

    SR = 8
    NSLAB = PW // SR
    NBUF = 4
    slabs = (gb0, gb1, gb2, gb3)
    sems = (sem_a, sem_b, sem_c, sem_d)

    def slab_copy(g, slot, hh):
        return pltpu.make_async_copy(
            dist_hbm.at[b, pl.ds(n0 + g * SR, SR), pl.ds(hh * (N // 2),
                                                         N // 2)],
            slabs[slot].at[:, pl.ds(hh * (N // 2), N // 2)], sems[slot])

    def slab_start(g, slot):
        slab_copy(g, slot, 0).start()
        slab_copy(g, slot, 1).start()

    def slab_wait(g, slot):
        slab_copy(g, slot, 0).wait()
        slab_copy(g, slot, 1).wait()

    for s in range(NBUF):
        slab_start(s, s)

    pltpu.sync_copy(idxk_hbm.at[pl.ds(base * K, PW * K)], idxk_v)
    pltpu.sync_copy(r_hbm.at[pl.ds(base, PW)], r_v)
    for c in range(C):
        pltpu.sync_copy(xt_hbm.at[pl.ds(c * BN + b * N, N)],
                        xs_v.at[pl.ds(c * N, N)])

    def mk_xr(t, carry):
        pvec = t * 16 + lane
        rvec = r_v[pl.ds(t * 16, 16)]
        g = plsc.load_gather(idxk_v, [pvec * K + rvec])
        for c in range(C):
            xr_v[pl.ds(c * PW + t * 16, 16)] = plsc.load_gather(
                xs_v, [g + c * N])
        return carry
    lax.fori_loop(0, PW // 16, mk_xr, 0, unroll=False)

    def process(g, slot, lane0, mv):
        sl = slabs[slot]
        for u in range(SR):
            p = g * SR + u
            iv0 = idxk_v[pl.ds(p * K, 16)]
            iv1 = idxk_v[pl.ds(p * K + 16, 16)]
            urow = jnp.full((16,), u, jnp.int32)
            a = plsc.load_gather(sl, [urow, iv0])
            bb = plsc.load_gather(sl, [urow, iv1])
            lo = jnp.minimum(jnp.sort(a), jnp.flip(jnp.sort(bb)))
            mv = jnp.where(lane == lane0 + u, jnp.max(lo), mv)
        return mv

    def quad(i, carry):
        mv = jnp.zeros((16,), jnp.float32)
        for q in range(NBUF):
            g = i * NBUF + q
            slab_wait(g, q)
            mv = process(g, q, 8 * (q % 2), mv)

            @pl.when(g + NBUF < NSLAB)
            def _():
                slab_start(g + NBUF, q)

            if q % 2 == 1:
                med_v[pl.ds((i * 2 + q // 2) * 16, 16)] = mv
                mv = jnp.zeros((16,), jnp.float32)
        return carry
    lax.fori_loop(0, NSLAB // NBUF, quad, 0, unroll=False)

    pltpu.sync_copy(med_v, med_hbm.at[pl.ds(base, PW)])
    for c in range(C):
        pltpu.sync_copy(xr_v.at[pl.ds(c * PW, PW)],
                        xr_hbm.at[pl.ds(c * BN + base, PW)])


def _sc_call(B, N, C, K, distance, idxk_flat, r_flat, xt_flat):
    BN = B * N
    PW = BN // NW
    mesh = plsc.VectorSubcoreMesh(core_axis_name="c", subcore_axis_name="s")
    kern = pl.kernel(
        functools.partial(_sc_body, B, N, C, K),
        out_type=(
            jax.ShapeDtypeStruct((BN,), jnp.float32),
            jax.ShapeDtypeStruct((C * BN,), jnp.float32),
        ),
        mesh=mesh,
        compiler_params=pltpu.CompilerParams(needs_layout_passes=False),
        scratch_types=[
            pltpu.VMEM((PW * K,), jnp.int32),
            pltpu.VMEM((8, N), jnp.float32),
            pltpu.VMEM((8, N), jnp.float32),
            pltpu.VMEM((8, N), jnp.float32),
            pltpu.VMEM((8, N), jnp.float32),
            pltpu.VMEM((C * N,), jnp.float32),
            pltpu.VMEM((PW,), jnp.int32),
            pltpu.VMEM((PW,), jnp.float32),
            pltpu.VMEM((C * PW,), jnp.float32),
            pltpu.SemaphoreType.DMA,
            pltpu.SemaphoreType.DMA,
            pltpu.SemaphoreType.DMA,
            pltpu.SemaphoreType.DMA,
        ],
    )
    return kern(distance, idxk_flat, r_flat, xt_flat)


def _tc_body(xt_ref, nt_ref, xr_ref, med_ref, out_ref):
    xt = xt_ref[...]
    nt = nt_ref[...]
    xv = xr_ref[...] - xt
    dot = jnp.sum(xv * nt, axis=0, keepdims=True)
    xp = xv - dot * nt
    n2 = jnp.sum(xp * xp, axis=0, keepdims=True)
    norms = jnp.maximum(jnp.sqrt(n2), 1e-6)
    half = norms * 0.5
    mk = jnp.sqrt(med_ref[...])
    clamped = jnp.where(half > mk, mk, half)
    out_ref[...] = xt + xp * (clamped / norms)


def _tc_call(xt, nt, xr, med):
    C, BN = xt.shape
    return pl.pallas_call(
        _tc_body,
        out_shape=jax.ShapeDtypeStruct((C, BN), jnp.float32),
    )(xt, nt, xr, med)


def kernel(x, distance, idx_k, normals):
    B, N, C = x.shape
    K = idx_k.shape[-1]
    BN = B * N
    r = jax.random.randint(jax.random.key(42), (B, N, 1), 0, K,
                           dtype=jnp.int32)
    xt = x.transpose(2, 0, 1).reshape(C, BN)
    nt = normals.transpose(2, 0, 1).reshape(C, BN)
    med, xr = _sc_call(B, N, C, K, distance, idx_k.reshape(BN * K),
                       r.reshape(BN), xt.reshape(C * BN))
    out_t = _tc_call(xt, nt, xr.reshape(C, BN), med.reshape(1, BN))
    return out_t.reshape(C, B, N).transpose(1, 2, 0)
